# baseline (device time: 126137 ns/iter reference)
import jax
import jax.numpy as jnp
from jax import lax
from jax.experimental import pallas as pl
from jax.experimental.pallas import tpu as pltpu

KY = 16
KL = 16
NF = 4


def kernel(x):
    m, n = x.shape
    n_half = n // 2
    h = m // 2
    r = h // KY
    rl = m // KL

    def body(
        x_ref, out_ref,
        yf32, lf32, ybuf, lbuf,
        ypack_sems, ysend_sems, yrecv_sems,
        xsend_sems, xrecv_sems,
        lpack_sems, lout_sems,
    ):
        my_x = lax.axis_index("x")
        my_y = lax.axis_index("y")
        y_nbr = (my_x, 1 - my_y)
        x_nbr = (1 - my_x, my_y)

        def ypack(k):
            p = pltpu.make_async_copy(
                x_ref.at[
                    pl.ds(my_x * h + k * r, r),
                    pl.ds((1 - my_y) * n_half, n_half),
                ],
                yf32.at[k % NF],
                ypack_sems.at[k % NF],
            )
            p.start()
            return p

        def lpack(j):
            p = pltpu.make_async_copy(
                x_ref.at[pl.ds(j * rl, rl), pl.ds(my_y * n_half, n_half)],
                lf32.at[j % NF],
                lpack_sems.at[j % NF],
            )
            p.start()
            return p

        ypacks = {k: ypack(k) for k in range(NF)}
        lpacks = {j: lpack(j) for j in range(NF)}
        ypacks[0].wait()
        ybuf[0, :, :] = yf32[0, :, :].astype(jnp.bfloat16)
        if NF < KY:
            ypacks[NF] = ypack(NF)

        barrier = pltpu.get_barrier_semaphore()
        for nbr in (y_nbr, x_nbr):
            pl.semaphore_signal(
                barrier, inc=1, device_id=nbr,
                device_id_type=pl.DeviceIdType.MESH,
            )
        pl.semaphore_wait(barrier, 2)

        y_rdmas = []
        for k in range(KY):
            rd = pltpu.make_async_remote_copy(
                src_ref=ybuf.at[k],
                dst_ref=out_ref.at[pl.ds(my_y * m + my_x * h + k * r, r), :],
                send_sem=ysend_sems.at[k],
                recv_sem=yrecv_sems.at[k],
                device_id=y_nbr,
                device_id_type=pl.DeviceIdType.MESH,
            )
            rd.start()
            y_rdmas.append(rd)
            j = k + 1
            if j < KY:
                ypacks[j].wait()
                ybuf[j, :, :] = yf32[j % NF, :, :].astype(jnp.bfloat16)
                if j + NF < KY:
                    ypacks[j + NF] = ypack(j + NF)

        louts = []
        x_rdmas = []
        for k in range(KY):
            rows = (1 - my_y) * m + my_x * h + k * r
            recv = pltpu.make_async_remote_copy(
                src_ref=ybuf.at[k],
                dst_ref=out_ref.at[pl.ds(rows, r), :],
                send_sem=ysend_sems.at[k],
                recv_sem=yrecv_sems.at[k],
                device_id=y_nbr,
                device_id_type=pl.DeviceIdType.MESH,
            )
            recv.wait_recv()
            fw = pltpu.make_async_remote_copy(
                src_ref=out_ref.at[pl.ds(rows, r), :],
                dst_ref=out_ref.at[pl.ds(rows, r), :],
                send_sem=xsend_sems.at[k],
                recv_sem=xrecv_sems.at[k],
                device_id=x_nbr,
                device_id_type=pl.DeviceIdType.MESH,
            )
            fw.start()
            x_rdmas.append(fw)
            for j in range(k * KL // KY, (k + 1) * KL // KY):
                lpacks[j].wait()
                lbuf[j, :, :] = lf32[j % NF, :, :].astype(jnp.bfloat16)
                if j + NF < KL:
                    lpacks[j + NF] = lpack(j + NF)
                o = pltpu.make_async_copy(
                    lbuf.at[j],
                    out_ref.at[pl.ds(my_y * m + j * rl, rl), :],
                    lout_sems.at[j],
                )
                o.start()
                louts.append(o)

        for rd in y_rdmas:
            rd.wait_send()
        for fw in x_rdmas:
            fw.wait_send()
        for o in louts:
            o.wait()
        for k in range(KY):
            rows = (1 - my_y) * m + (1 - my_x) * h + k * r
            recv = pltpu.make_async_remote_copy(
                src_ref=out_ref.at[pl.ds(rows, r), :],
                dst_ref=out_ref.at[pl.ds(rows, r), :],
                send_sem=xsend_sems.at[k],
                recv_sem=xrecv_sems.at[k],
                device_id=x_nbr,
                device_id_type=pl.DeviceIdType.MESH,
            )
            recv.wait_recv()

    return pl.pallas_call(
        body,
        out_shape=jax.ShapeDtypeStruct((2 * m, n_half), jnp.bfloat16),
        in_specs=[pl.BlockSpec(memory_space=pltpu.MemorySpace.HBM)],
        out_specs=pl.BlockSpec(memory_space=pltpu.MemorySpace.HBM),
        scratch_shapes=[
            pltpu.VMEM((NF, h // KY, n_half), jnp.float32),
            pltpu.VMEM((NF, m // KL, n_half), jnp.float32),
            pltpu.VMEM((KY, h // KY, n_half), jnp.bfloat16),
            pltpu.VMEM((KL, m // KL, n_half), jnp.bfloat16),
            pltpu.SemaphoreType.DMA((NF,)),
            pltpu.SemaphoreType.DMA((KY,)),
            pltpu.SemaphoreType.DMA((KY,)),
            pltpu.SemaphoreType.DMA((KY,)),
            pltpu.SemaphoreType.DMA((KY,)),
            pltpu.SemaphoreType.DMA((NF,)),
            pltpu.SemaphoreType.DMA((KL,)),
        ],
        compiler_params=pltpu.CompilerParams(
            collective_id=0, vmem_limit_bytes=56 * 1024 * 1024
        ),
    )(x)


# device time: 125446 ns/iter; 1.0055x vs baseline; 1.0055x over previous
import jax
import jax.numpy as jnp
from jax import lax
from jax.experimental import pallas as pl
from jax.experimental.pallas import tpu as pltpu

KY = 16
KL = 16
NF = 4


def kernel(x):
    m, n = x.shape
    n_half = n // 2
    h = m // 2
    r = h // KY
    rl = m // KL

    def body(
        x_ref, out_ref,
        yf32, lf32, ybuf, rbuf, lbuf,
        ypack_sems, ysend_sems, yrecv_sems,
        xsend_sems, xrecv_sems,
        store_sems, lpack_sems, lout_sems,
    ):
        my_x = lax.axis_index("x")
        my_y = lax.axis_index("y")
        y_nbr = (my_x, 1 - my_y)
        x_nbr = (1 - my_x, my_y)

        def ypack(k):
            p = pltpu.make_async_copy(
                x_ref.at[
                    pl.ds(my_x * h + k * r, r),
                    pl.ds((1 - my_y) * n_half, n_half),
                ],
                yf32.at[k % NF],
                ypack_sems.at[k % NF],
            )
            p.start()
            return p

        def lpack(j):
            p = pltpu.make_async_copy(
                x_ref.at[pl.ds(j * rl, rl), pl.ds(my_y * n_half, n_half)],
                lf32.at[j % NF],
                lpack_sems.at[j % NF],
            )
            p.start()
            return p

        ypacks = {k: ypack(k) for k in range(NF)}
        lpacks = {j: lpack(j) for j in range(NF)}
        ypacks[0].wait()
        ybuf[0, :, :] = yf32[0, :, :].astype(jnp.bfloat16)
        if NF < KY:
            ypacks[NF] = ypack(NF)

        barrier = pltpu.get_barrier_semaphore()
        for nbr in (y_nbr, x_nbr):
            pl.semaphore_signal(
                barrier, inc=1, device_id=nbr,
                device_id_type=pl.DeviceIdType.MESH,
            )
        pl.semaphore_wait(barrier, 2)

        y_rdmas = []
        for k in range(KY):
            rd = pltpu.make_async_remote_copy(
                src_ref=ybuf.at[k],
                dst_ref=rbuf.at[k],
                send_sem=ysend_sems.at[k],
                recv_sem=yrecv_sems.at[k],
                device_id=y_nbr,
                device_id_type=pl.DeviceIdType.MESH,
            )
            rd.start()
            y_rdmas.append(rd)
            j = k + 1
            if j < KY:
                ypacks[j].wait()
                ybuf[j, :, :] = yf32[j % NF, :, :].astype(jnp.bfloat16)
                if j + NF < KY:
                    ypacks[j + NF] = ypack(j + NF)

        louts = []
        stores = []
        x_rdmas = []
        for k in range(KY):
            recv = pltpu.make_async_remote_copy(
                src_ref=ybuf.at[k],
                dst_ref=rbuf.at[k],
                send_sem=ysend_sems.at[k],
                recv_sem=yrecv_sems.at[k],
                device_id=y_nbr,
                device_id_type=pl.DeviceIdType.MESH,
            )
            recv.wait_recv()
            rows = (1 - my_y) * m + my_x * h + k * r
            fw = pltpu.make_async_remote_copy(
                src_ref=rbuf.at[k],
                dst_ref=out_ref.at[pl.ds(rows, r), :],
                send_sem=xsend_sems.at[k],
                recv_sem=xrecv_sems.at[k],
                device_id=x_nbr,
                device_id_type=pl.DeviceIdType.MESH,
            )
            fw.start()
            x_rdmas.append(fw)
            st = pltpu.make_async_copy(
                rbuf.at[k],
                out_ref.at[pl.ds(rows, r), :],
                store_sems.at[k],
            )
            st.start()
            stores.append(st)
            for j in range(k * KL // KY, (k + 1) * KL // KY):
                lpacks[j].wait()
                lbuf[j, :, :] = lf32[j % NF, :, :].astype(jnp.bfloat16)
                if j + NF < KL:
                    lpacks[j + NF] = lpack(j + NF)
                o = pltpu.make_async_copy(
                    lbuf.at[j],
                    out_ref.at[pl.ds(my_y * m + j * rl, rl), :],
                    lout_sems.at[j],
                )
                o.start()
                louts.append(o)

        for rd in y_rdmas:
            rd.wait_send()
        for fw in x_rdmas:
            fw.wait_send()
        for st in stores:
            st.wait()
        for o in louts:
            o.wait()
        for k in range(KY):
            rows = (1 - my_y) * m + (1 - my_x) * h + k * r
            recv = pltpu.make_async_remote_copy(
                src_ref=rbuf.at[k],
                dst_ref=out_ref.at[pl.ds(rows, r), :],
                send_sem=xsend_sems.at[k],
                recv_sem=xrecv_sems.at[k],
                device_id=x_nbr,
                device_id_type=pl.DeviceIdType.MESH,
            )
            recv.wait_recv()

    return pl.pallas_call(
        body,
        out_shape=jax.ShapeDtypeStruct((2 * m, n_half), jnp.bfloat16),
        in_specs=[pl.BlockSpec(memory_space=pltpu.MemorySpace.HBM)],
        out_specs=pl.BlockSpec(memory_space=pltpu.MemorySpace.HBM),
        scratch_shapes=[
            pltpu.VMEM((NF, h // KY, n_half), jnp.float32),
            pltpu.VMEM((NF, m // KL, n_half), jnp.float32),
            pltpu.VMEM((KY, h // KY, n_half), jnp.bfloat16),
            pltpu.VMEM((KY, h // KY, n_half), jnp.bfloat16),
            pltpu.VMEM((KL, m // KL, n_half), jnp.bfloat16),
            pltpu.SemaphoreType.DMA((NF,)),
            pltpu.SemaphoreType.DMA((KY,)),
            pltpu.SemaphoreType.DMA((KY,)),
            pltpu.SemaphoreType.DMA((KY,)),
            pltpu.SemaphoreType.DMA((KY,)),
            pltpu.SemaphoreType.DMA((KY,)),
            pltpu.SemaphoreType.DMA((NF,)),
            pltpu.SemaphoreType.DMA((KL,)),
        ],
        compiler_params=pltpu.CompilerParams(
            collective_id=0, vmem_limit_bytes=56 * 1024 * 1024
        ),
    )(x)
